# baseline (speedup 1.0000x reference)
import jax
import jax.numpy as jnp
from jax import lax
from jax.experimental import pallas as pl
from jax.experimental.pallas import tpu as pltpu
from jax.experimental.pallas import tpu_sc as plsc

NUM_EMBEDDINGS = 1000000
FEATURES = 32
BATCH = 16384
SEQ = 20

NW = 32
BW = BATCH // NW
NSTREAM = BW // 128
FT = FEATURES // 8
NBUF = 2


def _body(idx_hbm, table_hbm, out_hbm, idx_v, rows_v, tiles_v, gsem, isem):
    wid = lax.axis_index("s") * 2 + lax.axis_index("c")
    b0 = wid * BW

    def start_idx(s, b):
        pltpu.async_copy(idx_hbm.at[s, pl.ds(b0, BW)], idx_v.at[b], isem.at[b])

    def wait_idx(s, b):
        pltpu.make_async_copy(idx_hbm.at[s, pl.ds(b0, BW)], idx_v.at[b],
                              isem.at[b]).wait()

    def start_gathers(b):
        for k in range(NSTREAM):
            pltpu.async_copy(table_hbm.at[idx_v.at[b, pl.ds(k * 128, 128)]],
                             rows_v.at[b, pl.ds(k * 128, 128)], gsem.at[b])

    def wait_gathers(b):
        for k in range(NSTREAM):
            pltpu.make_async_copy(
                table_hbm.at[idx_v.at[b, pl.ds(k * 128, 128)]],
                rows_v.at[b, pl.ds(k * 128, 128)], gsem.at[b]).wait()

    lanes = lax.iota(jnp.int32, 16)

    def transpose_and_write(s, b):
        def tc_step(tc, _):
            t = tc // NSTREAM
            c = tc % NSTREAM
            for jm in range(8):
                j = jnp.broadcast_to(t * 8 + jm, (16,)).astype(jnp.int32)
                for g in range(8):
                    rows = lanes + (c * 128 + g * 16)
                    vals = plsc.load_gather(rows_v.at[b], [rows, j])
                    tiles_v[t, c, jm, pl.ds(g * 16, 16)] = vals
            return ()

        lax.fori_loop(0, FT * NSTREAM, tc_step, ())
        for t in range(FT):
            pltpu.sync_copy(tiles_v.at[t],
                            out_hbm.at[s, t, pl.ds(wid * NSTREAM, NSTREAM)])

    for b in range(NBUF):
        start_idx(b, b)
        wait_idx(b, b)
        start_gathers(b)

    def group(g, _):
        for b in range(NBUF):
            s = g * NBUF + b
            wait_gathers(b)
            start_idx(s + NBUF, b)
            transpose_and_write(s, b)
            wait_idx(s + NBUF, b)
            start_gathers(b)
        return ()

    lax.fori_loop(0, (SEQ - NBUF) // NBUF, group, ())

    for b in range(NBUF):
        s = SEQ - NBUF + b
        wait_gathers(b)
        transpose_and_write(s, b)


def kernel(inputs, embedding):
    idx_t = inputs.T
    mesh = plsc.VectorSubcoreMesh(core_axis_name="c", subcore_axis_name="s")
    out = pl.kernel(
        _body,
        mesh=mesh,
        out_type=jax.ShapeDtypeStruct((SEQ, FT, BATCH // 128, 8, 128),
                                      jnp.float32),
        scratch_types=[
            pltpu.VMEM((NBUF, BW), jnp.int32),
            pltpu.VMEM((NBUF, BW, FEATURES), jnp.float32),
            pltpu.VMEM((FT, NSTREAM, 8, 128), jnp.float32),
            pltpu.SemaphoreType.DMA((NBUF,)),
            pltpu.SemaphoreType.DMA((NBUF,)),
        ],
        compiler_params=pltpu.CompilerParams(use_tc_tiling_on_sc=False,
                                             needs_layout_passes=False),
    )(idx_t, embedding)
    return out.transpose(2, 4, 0, 1, 3).reshape(BATCH, SEQ, FEATURES)


# speedup vs baseline: 1.1046x; 1.1046x over previous
import jax
import jax.numpy as jnp
from jax import lax
from jax.experimental import pallas as pl
from jax.experimental.pallas import tpu as pltpu
from jax.experimental.pallas import tpu_sc as plsc

NUM_EMBEDDINGS = 1000000
FEATURES = 32
BATCH = 16384
SEQ = 20

NW = 32
BW = BATCH // NW
NSTREAM = BW // 128
FT = FEATURES // 8
NBUF = 2


def _body(idx_hbm, table_hbm, out_hbm, idx_v, rows_v, tiles_v, gsem, isem):
    wid = lax.axis_index("s") * 2 + lax.axis_index("c")
    b0 = wid * BW

    def start_idx(s, b):
        pltpu.async_copy(idx_hbm.at[s, pl.ds(b0, BW)], idx_v.at[b], isem.at[b])

    def wait_idx(s, b):
        pltpu.make_async_copy(idx_hbm.at[s, pl.ds(b0, BW)], idx_v.at[b],
                              isem.at[b]).wait()

    def start_gathers(b):
        for k in range(NSTREAM):
            pltpu.async_copy(table_hbm.at[idx_v.at[b, pl.ds(k * 128, 128)]],
                             rows_v.at[b, pl.ds(k * 128, 128)], gsem.at[b])

    def wait_gathers(b):
        for k in range(NSTREAM):
            pltpu.make_async_copy(
                table_hbm.at[idx_v.at[b, pl.ds(k * 128, 128)]],
                rows_v.at[b, pl.ds(k * 128, 128)], gsem.at[b]).wait()

    lanes = lax.iota(jnp.int32, 16)

    def transpose_and_write(s, b):
        @plsc.parallel_loop(0, FT * NSTREAM, unroll=2)
        def tc_step(tc):
            t = tc // NSTREAM
            c = tc % NSTREAM
            rows_c = lanes + c * 128
            for jm in range(8):
                j = jnp.broadcast_to(t * 8 + jm, (16,)).astype(jnp.int32)
                for g in range(8):
                    vals = plsc.load_gather(rows_v.at[b], [rows_c + g * 16, j])
                    tiles_v[t, c, jm, pl.ds(g * 16, 16)] = vals
        for t in range(FT):
            pltpu.sync_copy(tiles_v.at[t],
                            out_hbm.at[s, t, pl.ds(wid * NSTREAM, NSTREAM)])

    for b in range(NBUF):
        start_idx(b, b)
        wait_idx(b, b)
        start_gathers(b)

    def group(g, _):
        for b in range(NBUF):
            s = g * NBUF + b
            wait_gathers(b)
            start_idx(s + NBUF, b)
            transpose_and_write(s, b)
            wait_idx(s + NBUF, b)
            start_gathers(b)
        return ()

    lax.fori_loop(0, (SEQ - NBUF) // NBUF, group, ())

    for b in range(NBUF):
        s = SEQ - NBUF + b
        wait_gathers(b)
        transpose_and_write(s, b)


def kernel(inputs, embedding):
    idx_t = inputs.T
    mesh = plsc.VectorSubcoreMesh(core_axis_name="c", subcore_axis_name="s")
    out = pl.kernel(
        _body,
        mesh=mesh,
        out_type=jax.ShapeDtypeStruct((SEQ, FT, BATCH // 128, 8, 128),
                                      jnp.float32),
        scratch_types=[
            pltpu.VMEM((NBUF, BW), jnp.int32),
            pltpu.VMEM((NBUF, BW, FEATURES), jnp.float32),
            pltpu.VMEM((FT, NSTREAM, 8, 128), jnp.float32),
            pltpu.SemaphoreType.DMA((NBUF,)),
            pltpu.SemaphoreType.DMA((NBUF,)),
        ],
        compiler_params=pltpu.CompilerParams(use_tc_tiling_on_sc=False,
                                             needs_layout_passes=False),
    )(idx_t, embedding)
    return out.transpose(2, 4, 0, 1, 3).reshape(BATCH, SEQ, FEATURES)
